# baseline (device time: 46994 ns/iter reference)
import jax
import jax.numpy as jnp
from jax import lax
from jax.experimental import pallas as pl
from jax.experimental.pallas import tpu as pltpu

B, S, H, D = 4, 512, 8, 64
K = H * D
N = 1024
SH = S // 2


def _head_matmul(o_block, w, h0=0):
    acc = None
    for h in range(H):
        part = jnp.dot(o_block[:, h, :], w[(h0 + h) * D:(h0 + h + 1) * D, :],
                       preferred_element_type=jnp.float32)
        acc = part if acc is None else acc + part
    return acc


def kernel(O, Wo):
    def body(o_hbm, w_hbm, out_hbm,
             w_vmem, o_vmem, out_vmem,
             send_wo, recv_wo, send_o, recv_o,
             w_in_sem, o_in_sems, out_sems, wo_sems, o_sems):
        my_x = lax.axis_index("x")
        my_y = lax.axis_index("y")
        my_z = lax.axis_index("z")
        other = 1 - my_x
        nbr = (other, my_y, my_z)

        w_in = pltpu.make_async_copy(w_hbm, w_vmem, w_in_sem)
        w_in.start()
        o_ins = []
        for b in range(B):
            cp = pltpu.make_async_copy(o_hbm.at[b], o_vmem.at[b],
                                       o_in_sems.at[b])
            cp.start()
            o_ins.append(cp)

        barrier = pltpu.get_barrier_semaphore()
        pl.semaphore_signal(
            barrier, inc=1,
            device_id=nbr, device_id_type=pl.DeviceIdType.MESH,
        )
        pl.semaphore_wait(barrier, 1)

        w_in.wait()
        w = w_vmem[...].astype(jnp.bfloat16)
        send_wo[...] = w
        wo_rdma = pltpu.make_async_remote_copy(
            src_ref=send_wo, dst_ref=recv_wo,
            send_sem=wo_sems.at[0], recv_sem=wo_sems.at[1],
            device_id=nbr, device_id_type=pl.DeviceIdType.MESH,
        )
        wo_rdma.start()

        o_rdmas = []
        for b in range(B):
            o_ins[b].wait()
            send_o[b, :, :, :] = o_vmem[
                b, pl.ds(other * SH, SH), :, :].astype(jnp.bfloat16)
            rdma = pltpu.make_async_remote_copy(
                src_ref=send_o.at[b], dst_ref=recv_o.at[b],
                send_sem=o_sems.at[0, b], recv_sem=o_sems.at[1, b],
                device_id=nbr, device_id_type=pl.DeviceIdType.MESH,
            )
            rdma.start()
            o_rdmas.append(rdma)

        for b in range(B):
            mine = o_vmem[b, pl.ds(my_x * SH, SH), :, :].astype(jnp.bfloat16)
            out_vmem[b, :, :] = _head_matmul(mine, w)

        wo_rdma.wait_recv()
        rw = recv_wo[...]
        out_copies = []
        for b in range(B):
            o_rdmas[b].wait_recv()
            out_vmem[b, :, :] += _head_matmul(recv_o[b, :, :, :], rw)
            cp = pltpu.make_async_copy(
                out_vmem.at[b], out_hbm.at[b], out_sems.at[b])
            cp.start()
            out_copies.append(cp)

        for cp in out_copies:
            cp.wait()
        wo_rdma.wait_send()
        for b in range(B):
            o_rdmas[b].wait_send()

    return pl.pallas_call(
        body,
        out_shape=jax.ShapeDtypeStruct((B, SH, N), jnp.float32),
        in_specs=[
            pl.BlockSpec(memory_space=pl.ANY),
            pl.BlockSpec(memory_space=pl.ANY),
        ],
        out_specs=pl.BlockSpec(memory_space=pl.ANY),
        scratch_shapes=[
            pltpu.VMEM((K, N), jnp.float32),
            pltpu.VMEM((B, S, H, D), jnp.float32),
            pltpu.VMEM((B, SH, N), jnp.float32),
            pltpu.VMEM((K, N), jnp.bfloat16),
            pltpu.VMEM((K, N), jnp.bfloat16),
            pltpu.VMEM((B, SH, H, D), jnp.bfloat16),
            pltpu.VMEM((B, SH, H, D), jnp.bfloat16),
            pltpu.SemaphoreType.DMA,
            pltpu.SemaphoreType.DMA((B,)),
            pltpu.SemaphoreType.DMA((B,)),
            pltpu.SemaphoreType.DMA((2,)),
            pltpu.SemaphoreType.DMA((2, B)),
        ],
        compiler_params=pltpu.CompilerParams(collective_id=0),
    )(O, Wo)


# device time: 34872 ns/iter; 1.3476x vs baseline; 1.3476x over previous
import jax
import jax.numpy as jnp
from jax import lax
from jax.experimental import pallas as pl
from jax.experimental.pallas import tpu as pltpu

B, S, H, D = 4, 512, 8, 64
K = H * D
N = 1024
SH = S // 2


def kernel(O, Wo):
    def body(o_hbm, w_hbm, out_hbm,
             w_vmem, o_vmem, out_vmem,
             send_wo, recv_wo, send_o, recv_o,
             w_in_sem, o_in_sems, out_sems, wo_sems, o_sems):
        my_x = lax.axis_index("x")
        my_y = lax.axis_index("y")
        my_z = lax.axis_index("z")
        other = 1 - my_x
        nbr = (other, my_y, my_z)

        w_in = pltpu.make_async_copy(w_hbm, w_vmem, w_in_sem)
        w_in.start()
        o_ins = []
        for b in range(B):
            cp = pltpu.make_async_copy(o_hbm.at[b], o_vmem.at[b],
                                       o_in_sems.at[b])
            cp.start()
            o_ins.append(cp)

        barrier = pltpu.get_barrier_semaphore()
        pl.semaphore_signal(
            barrier, inc=1,
            device_id=nbr, device_id_type=pl.DeviceIdType.MESH,
        )
        pl.semaphore_wait(barrier, 1)

        w_in.wait()
        w = w_vmem[...].astype(jnp.bfloat16)
        send_wo[...] = w
        wo_rdma = pltpu.make_async_remote_copy(
            src_ref=send_wo, dst_ref=recv_wo,
            send_sem=wo_sems.at[0], recv_sem=wo_sems.at[1],
            device_id=nbr, device_id_type=pl.DeviceIdType.MESH,
        )
        wo_rdma.start()

        o_rdmas = []
        for b in range(B):
            row = pl.ds(b * SH, SH)
            o_ins[b].wait()
            chunk = o_vmem[b, pl.ds(other * SH, SH), :, :].astype(jnp.bfloat16)
            send_o[row, :] = chunk.reshape(SH, K)
            rdma = pltpu.make_async_remote_copy(
                src_ref=send_o.at[row, :], dst_ref=recv_o.at[row, :],
                send_sem=o_sems.at[0, b], recv_sem=o_sems.at[1, b],
                device_id=nbr, device_id_type=pl.DeviceIdType.MESH,
            )
            rdma.start()
            o_rdmas.append(rdma)

        for b in range(B):
            mine = o_vmem[b, pl.ds(my_x * SH, SH), :, :].astype(jnp.bfloat16)
            out_vmem[b, :, :] = jnp.dot(mine.reshape(SH, K), w,
                                        preferred_element_type=jnp.float32)

        wo_rdma.wait_recv()
        rw = recv_wo[...]
        out_copies = []
        for b in range(B):
            row = pl.ds(b * SH, SH)
            o_rdmas[b].wait_recv()
            out_vmem[b, :, :] += jnp.dot(recv_o[row, :], rw,
                                         preferred_element_type=jnp.float32)
            cp = pltpu.make_async_copy(
                out_vmem.at[b], out_hbm.at[b], out_sems.at[b])
            cp.start()
            out_copies.append(cp)

        for cp in out_copies:
            cp.wait()
        wo_rdma.wait_send()
        for b in range(B):
            o_rdmas[b].wait_send()

    return pl.pallas_call(
        body,
        out_shape=jax.ShapeDtypeStruct((B, SH, N), jnp.float32),
        in_specs=[
            pl.BlockSpec(memory_space=pl.ANY),
            pl.BlockSpec(memory_space=pl.ANY),
        ],
        out_specs=pl.BlockSpec(memory_space=pl.ANY),
        scratch_shapes=[
            pltpu.VMEM((K, N), jnp.float32),
            pltpu.VMEM((B, S, H, D), jnp.float32),
            pltpu.VMEM((B, SH, N), jnp.float32),
            pltpu.VMEM((K, N), jnp.bfloat16),
            pltpu.VMEM((K, N), jnp.bfloat16),
            pltpu.VMEM((B * SH, K), jnp.bfloat16),
            pltpu.VMEM((B * SH, K), jnp.bfloat16),
            pltpu.SemaphoreType.DMA,
            pltpu.SemaphoreType.DMA((B,)),
            pltpu.SemaphoreType.DMA((B,)),
            pltpu.SemaphoreType.DMA((2,)),
            pltpu.SemaphoreType.DMA((2, B)),
        ],
        compiler_params=pltpu.CompilerParams(collective_id=0),
    )(O, Wo)


# device time: 33192 ns/iter; 1.4158x vs baseline; 1.0506x over previous
import jax
import jax.numpy as jnp
from jax import lax
from jax.experimental import pallas as pl
from jax.experimental.pallas import tpu as pltpu

B, S, H, D = 4, 512, 8, 64
K = H * D
N = 1024
SH = S // 2

_TN = (((0,), (0,)), ((), ()))


def kernel(O, Wo):
    OT = O.transpose(0, 2, 3, 1)

    def body(ot_hbm, w_hbm, out_hbm,
             w_vmem, ot_vmem, out_vmem,
             send_wo, recv_wo, send_o, recv_o,
             w_in_sem, o_in_sems, out_sems, wo_sems, o_sems):
        my_x = lax.axis_index("x")
        my_y = lax.axis_index("y")
        my_z = lax.axis_index("z")
        other = 1 - my_x
        nbr = (other, my_y, my_z)

        w_in = pltpu.make_async_copy(w_hbm, w_vmem, w_in_sem)
        w_in.start()
        o_ins = []
        for b in range(B):
            cp = pltpu.make_async_copy(ot_hbm.at[b], ot_vmem.at[b],
                                       o_in_sems.at[b])
            cp.start()
            o_ins.append(cp)

        barrier = pltpu.get_barrier_semaphore()
        pl.semaphore_signal(
            barrier, inc=1,
            device_id=nbr, device_id_type=pl.DeviceIdType.MESH,
        )
        pl.semaphore_wait(barrier, 1)

        w_in.wait()
        w = w_vmem[...].astype(jnp.bfloat16)
        send_wo[...] = w
        wo_rdma = pltpu.make_async_remote_copy(
            src_ref=send_wo, dst_ref=recv_wo,
            send_sem=wo_sems.at[0], recv_sem=wo_sems.at[1],
            device_id=nbr, device_id_type=pl.DeviceIdType.MESH,
        )
        wo_rdma.start()

        o_rdmas = []
        for b in range(B):
            o_ins[b].wait()
            oth = ot_vmem[b, :, :, pl.ds(other * SH, SH)]
            send_o[b, :, :] = oth.astype(jnp.bfloat16).reshape(K, SH)
            rdma = pltpu.make_async_remote_copy(
                src_ref=send_o.at[b], dst_ref=recv_o.at[b],
                send_sem=o_sems.at[0, b], recv_sem=o_sems.at[1, b],
                device_id=nbr, device_id_type=pl.DeviceIdType.MESH,
            )
            rdma.start()
            o_rdmas.append(rdma)

        for b in range(B):
            mine = ot_vmem[b, :, :, pl.ds(my_x * SH, SH)]
            out_vmem[b, :, :] = lax.dot_general(
                mine.astype(jnp.bfloat16).reshape(K, SH), w, _TN,
                preferred_element_type=jnp.float32)

        wo_rdma.wait_recv()
        rw = recv_wo[...]
        out_copies = []
        for b in range(B):
            o_rdmas[b].wait_recv()
            out_vmem[b, :, :] += lax.dot_general(
                recv_o[b], rw, _TN, preferred_element_type=jnp.float32)
            cp = pltpu.make_async_copy(
                out_vmem.at[b], out_hbm.at[b], out_sems.at[b])
            cp.start()
            out_copies.append(cp)

        for cp in out_copies:
            cp.wait()
        wo_rdma.wait_send()
        for b in range(B):
            o_rdmas[b].wait_send()

    return pl.pallas_call(
        body,
        out_shape=jax.ShapeDtypeStruct((B, SH, N), jnp.float32),
        in_specs=[
            pl.BlockSpec(memory_space=pl.ANY),
            pl.BlockSpec(memory_space=pl.ANY),
        ],
        out_specs=pl.BlockSpec(memory_space=pl.ANY),
        scratch_shapes=[
            pltpu.VMEM((K, N), jnp.float32),
            pltpu.VMEM((B, H, D, S), jnp.float32),
            pltpu.VMEM((B, SH, N), jnp.float32),
            pltpu.VMEM((K, N), jnp.bfloat16),
            pltpu.VMEM((K, N), jnp.bfloat16),
            pltpu.VMEM((B, K, SH), jnp.bfloat16),
            pltpu.VMEM((B, K, SH), jnp.bfloat16),
            pltpu.SemaphoreType.DMA,
            pltpu.SemaphoreType.DMA((B,)),
            pltpu.SemaphoreType.DMA((B,)),
            pltpu.SemaphoreType.DMA((2,)),
            pltpu.SemaphoreType.DMA((2, B)),
        ],
        compiler_params=pltpu.CompilerParams(collective_id=0),
    )(OT, Wo)


# device time: 32514 ns/iter; 1.4453x vs baseline; 1.0209x over previous
import jax
import jax.numpy as jnp
from jax import lax
from jax.experimental import pallas as pl
from jax.experimental.pallas import tpu as pltpu

B, S, H, D = 4, 512, 8, 64
K = H * D
N = 1024
SH = S // 2

_TN = (((0,), (0,)), ((), ()))


def kernel(O, Wo):
    OT = O.transpose(0, 2, 3, 1)

    def body(ot_hbm, w_hbm, out_hbm,
             w_vmem, ot_vmem, out_vmem, out_bf,
             send_wo, recv_wo, send_o, recv_o,
             w_in_sem, o_in_sems, out_sems, wo_sems, o_sems):
        my_x = lax.axis_index("x")
        my_y = lax.axis_index("y")
        my_z = lax.axis_index("z")
        other = 1 - my_x
        nbr = (other, my_y, my_z)

        w_in = pltpu.make_async_copy(w_hbm, w_vmem, w_in_sem)
        w_in.start()
        o_ins = []
        for b in range(B):
            cp = pltpu.make_async_copy(ot_hbm.at[b], ot_vmem.at[b],
                                       o_in_sems.at[b])
            cp.start()
            o_ins.append(cp)

        barrier = pltpu.get_barrier_semaphore()
        pl.semaphore_signal(
            barrier, inc=1,
            device_id=nbr, device_id_type=pl.DeviceIdType.MESH,
        )
        pl.semaphore_wait(barrier, 1)

        w_in.wait()
        w = w_vmem[...].astype(jnp.bfloat16)
        send_wo[...] = w
        wo_rdma = pltpu.make_async_remote_copy(
            src_ref=send_wo, dst_ref=recv_wo,
            send_sem=wo_sems.at[0], recv_sem=wo_sems.at[1],
            device_id=nbr, device_id_type=pl.DeviceIdType.MESH,
        )
        wo_rdma.start()

        o_rdmas = []
        for b in range(B):
            o_ins[b].wait()
            oth = ot_vmem[b, :, :, pl.ds(other * SH, SH)]
            send_o[b, :, :] = oth.astype(jnp.bfloat16).reshape(K, SH)
            rdma = pltpu.make_async_remote_copy(
                src_ref=send_o.at[b], dst_ref=recv_o.at[b],
                send_sem=o_sems.at[0, b], recv_sem=o_sems.at[1, b],
                device_id=nbr, device_id_type=pl.DeviceIdType.MESH,
            )
            rdma.start()
            o_rdmas.append(rdma)

        for b in range(B):
            mine = ot_vmem[b, :, :, pl.ds(my_x * SH, SH)]
            out_vmem[b, :, :] = lax.dot_general(
                mine.astype(jnp.bfloat16).reshape(K, SH), w, _TN,
                preferred_element_type=jnp.float32)

        wo_rdma.wait_recv()
        rw = recv_wo[...]
        out_copies = []
        for b in range(B):
            o_rdmas[b].wait_recv()
            total = out_vmem[b, :, :] + lax.dot_general(
                recv_o[b], rw, _TN, preferred_element_type=jnp.float32)
            out_bf[b, :, :] = total.astype(jnp.bfloat16)
            cp = pltpu.make_async_copy(
                out_bf.at[b], out_hbm.at[b], out_sems.at[b])
            cp.start()
            out_copies.append(cp)

        for cp in out_copies:
            cp.wait()
        wo_rdma.wait_send()
        for b in range(B):
            o_rdmas[b].wait_send()

    return pl.pallas_call(
        body,
        out_shape=jax.ShapeDtypeStruct((B, SH, N), jnp.bfloat16),
        in_specs=[
            pl.BlockSpec(memory_space=pl.ANY),
            pl.BlockSpec(memory_space=pl.ANY),
        ],
        out_specs=pl.BlockSpec(memory_space=pl.ANY),
        scratch_shapes=[
            pltpu.VMEM((K, N), jnp.float32),
            pltpu.VMEM((B, H, D, S), jnp.float32),
            pltpu.VMEM((B, SH, N), jnp.float32),
            pltpu.VMEM((B, SH, N), jnp.bfloat16),
            pltpu.VMEM((K, N), jnp.bfloat16),
            pltpu.VMEM((K, N), jnp.bfloat16),
            pltpu.VMEM((B, K, SH), jnp.bfloat16),
            pltpu.VMEM((B, K, SH), jnp.bfloat16),
            pltpu.SemaphoreType.DMA,
            pltpu.SemaphoreType.DMA((B,)),
            pltpu.SemaphoreType.DMA((B,)),
            pltpu.SemaphoreType.DMA((2,)),
            pltpu.SemaphoreType.DMA((2, B)),
        ],
        compiler_params=pltpu.CompilerParams(collective_id=0),
    )(OT, Wo)


# device time: 26794 ns/iter; 1.7539x vs baseline; 1.2135x over previous
import jax
import jax.numpy as jnp
from jax import lax
from jax.experimental import pallas as pl
from jax.experimental.pallas import tpu as pltpu

B, S, H, D = 4, 512, 8, 64
K = H * D
N = 1024
SH = S // 2
NC = 4
KC = K // NC

_TN = (((0,), (0,)), ((), ()))


def kernel(O, Wo):
    OT = O.transpose(0, 2, 3, 1)

    def body(ot_hbm, w_hbm, out_hbm,
             w_vmem, ot_vmem, out_vmem, out_bf,
             send_wo, recv_wo, send_o, recv_o,
             w_in_sem, o_in_sems, out_sems,
             x_send_sems, wo_recv_sems, o_recv_sems, fwd_send_sems):
        my_x = lax.axis_index("x")
        my_y = lax.axis_index("y")
        my_z = lax.axis_index("z")
        xpeer = (1 - my_x, my_y, my_z)
        ypart = (my_x, my_y + 1 - 2 * (my_y % 2), my_z)
        is_even = my_y % 2 == 0

        w_in = pltpu.make_async_copy(w_hbm, w_vmem, w_in_sem)
        w_in.start()
        o_ins = []
        for b in range(B):
            cp = pltpu.make_async_copy(ot_hbm.at[b], ot_vmem.at[b],
                                       o_in_sems.at[b])
            cp.start()
            o_ins.append(cp)

        barrier = pltpu.get_barrier_semaphore()
        for nbr in (xpeer, ypart):
            pl.semaphore_signal(
                barrier, inc=1,
                device_id=nbr, device_id_type=pl.DeviceIdType.MESH,
            )
        pl.semaphore_wait(barrier, 2)

        w_in.wait()
        w = w_vmem[...].astype(jnp.bfloat16)
        send_wo[...] = w

        other = 1 - my_x
        for b in range(B):
            o_ins[b].wait()
            oth = ot_vmem[b, :, :, pl.ds(other * SH, SH)]
            send_o[b, :, :] = oth.astype(jnp.bfloat16).reshape(K, SH)

        wo_direct = [
            pltpu.make_async_remote_copy(
                src_ref=send_wo.at[pl.ds(i * KC, KC), :],
                dst_ref=recv_wo.at[pl.ds(i * KC, KC), :],
                send_sem=x_send_sems.at[i], recv_sem=wo_recv_sems.at[i],
                device_id=xpeer, device_id_type=pl.DeviceIdType.MESH,
            ) for i in range(NC)
        ]
        o_direct = [
            pltpu.make_async_remote_copy(
                src_ref=send_o.at[b], dst_ref=recv_o.at[b],
                send_sem=x_send_sems.at[b], recv_sem=o_recv_sems.at[b],
                device_id=xpeer, device_id_type=pl.DeviceIdType.MESH,
            ) for b in range(B)
        ]

        @pl.when(is_even)
        def _():
            for r in wo_direct:
                r.start()

        @pl.when(jnp.logical_not(is_even))
        def _():
            for r in o_direct:
                r.start()

        for b in range(B):
            mine = ot_vmem[b, :, :, pl.ds(my_x * SH, SH)]
            out_vmem[b, :, :] = lax.dot_general(
                mine.astype(jnp.bfloat16).reshape(K, SH), w, _TN,
                preferred_element_type=jnp.float32)

        wo_fwd = [
            pltpu.make_async_remote_copy(
                src_ref=recv_wo.at[pl.ds(i * KC, KC), :],
                dst_ref=recv_wo.at[pl.ds(i * KC, KC), :],
                send_sem=fwd_send_sems.at[i], recv_sem=wo_recv_sems.at[i],
                device_id=ypart, device_id_type=pl.DeviceIdType.MESH,
            ) for i in range(NC)
        ]
        o_fwd = [
            pltpu.make_async_remote_copy(
                src_ref=recv_o.at[b], dst_ref=recv_o.at[b],
                send_sem=fwd_send_sems.at[b], recv_sem=o_recv_sems.at[b],
                device_id=ypart, device_id_type=pl.DeviceIdType.MESH,
            ) for b in range(B)
        ]

        def tile(b, i):
            out_vmem[b, :, :] += lax.dot_general(
                recv_o[b, pl.ds(i * KC, KC), :],
                recv_wo[pl.ds(i * KC, KC), :],
                _TN, preferred_element_type=jnp.float32)

        def flush(b):
            out_bf[b, :, :] = out_vmem[b, :, :].astype(jnp.bfloat16)
            cp = pltpu.make_async_copy(
                out_bf.at[b], out_hbm.at[b], out_sems.at[b])
            cp.start()
            return cp

        @pl.when(is_even)
        def _():
            for i in range(NC):
                wo_direct[i].wait_recv()
                wo_fwd[i].start()
            for b in range(B):
                o_fwd[b].wait_recv()
                for i in range(NC):
                    tile(b, i)
                flush(b)

        @pl.when(jnp.logical_not(is_even))
        def _():
            for b in range(B):
                o_direct[b].wait_recv()
                o_fwd[b].start()
            for i in range(NC):
                wo_fwd[i].wait_recv()
                for b in range(B):
                    tile(b, i)
                    if i == NC - 1:
                        flush(b)

        def out_wait(b):
            pltpu.make_async_copy(
                out_bf.at[b], out_hbm.at[b], out_sems.at[b]).wait()

        @pl.when(is_even)
        def _():
            for i in range(NC):
                wo_direct[i].wait_send()
                wo_fwd[i].wait_send()
            for b in range(B):
                out_wait(b)

        @pl.when(jnp.logical_not(is_even))
        def _():
            for b in range(B):
                o_direct[b].wait_send()
                o_fwd[b].wait_send()
                out_wait(b)

    return pl.pallas_call(
        body,
        out_shape=jax.ShapeDtypeStruct((B, SH, N), jnp.bfloat16),
        in_specs=[
            pl.BlockSpec(memory_space=pl.ANY),
            pl.BlockSpec(memory_space=pl.ANY),
        ],
        out_specs=pl.BlockSpec(memory_space=pl.ANY),
        scratch_shapes=[
            pltpu.VMEM((K, N), jnp.float32),
            pltpu.VMEM((B, H, D, S), jnp.float32),
            pltpu.VMEM((B, SH, N), jnp.float32),
            pltpu.VMEM((B, SH, N), jnp.bfloat16),
            pltpu.VMEM((K, N), jnp.bfloat16),
            pltpu.VMEM((K, N), jnp.bfloat16),
            pltpu.VMEM((B, K, SH), jnp.bfloat16),
            pltpu.VMEM((B, K, SH), jnp.bfloat16),
            pltpu.SemaphoreType.DMA,
            pltpu.SemaphoreType.DMA((B,)),
            pltpu.SemaphoreType.DMA((B,)),
            pltpu.SemaphoreType.DMA((NC,)),
            pltpu.SemaphoreType.DMA((NC,)),
            pltpu.SemaphoreType.DMA((B,)),
            pltpu.SemaphoreType.DMA((NC,)),
        ],
        compiler_params=pltpu.CompilerParams(collective_id=0),
    )(OT, Wo)


# device time: 25569 ns/iter; 1.8379x vs baseline; 1.0479x over previous
import jax
import jax.numpy as jnp
from jax import lax
from jax.experimental import pallas as pl
from jax.experimental.pallas import tpu as pltpu

B, S, H, D = 4, 512, 8, 64
K = H * D
N = 1024
SH = S // 2
NC = 4
KC = K // NC
NW = 8
KW = K // NW
SW = SH // 2

_TN = (((0,), (0,)), ((), ()))


def kernel(O, Wo):
    OT = O.transpose(0, 2, 3, 1)

    def body(ot_hbm, w_hbm, out_hbm,
             w_vmem, ot_vmem, out_vmem, out_bf,
             send_wo, recv_wo, send_o, recv_o,
             w_in_sem, o_in_sems, out_sems,
             x_send_sems, wo_recv_sems, o_recv_sems, fwd_send_sems):
        my_x = lax.axis_index("x")
        my_y = lax.axis_index("y")
        my_z = lax.axis_index("z")
        xpeer = (1 - my_x, my_y, my_z)
        ypart = (my_x, my_y + 1 - 2 * (my_y % 2), my_z)
        is_even = my_y % 2 == 0

        w_in = pltpu.make_async_copy(w_hbm, w_vmem, w_in_sem)
        w_in.start()
        o_ins = []
        for b in range(B):
            cp = pltpu.make_async_copy(ot_hbm.at[b], ot_vmem.at[b],
                                       o_in_sems.at[b])
            cp.start()
            o_ins.append(cp)

        barrier = pltpu.get_barrier_semaphore()
        for nbr in (xpeer, ypart):
            pl.semaphore_signal(
                barrier, inc=1,
                device_id=nbr, device_id_type=pl.DeviceIdType.MESH,
            )
        pl.semaphore_wait(barrier, 2)

        w_in.wait()
        w = w_vmem[...].astype(jnp.bfloat16)
        send_wo[...] = w

        other = 1 - my_x
        for b in range(B):
            o_ins[b].wait()
            oth = ot_vmem[b, :, :, pl.ds(other * SH, SH)]
            send_o[b, :, :] = oth.astype(jnp.bfloat16).reshape(K, SH)

        wo_direct = [
            pltpu.make_async_remote_copy(
                src_ref=send_wo.at[pl.ds(j * KW, KW), :],
                dst_ref=recv_wo.at[pl.ds(j * KW, KW), :],
                send_sem=x_send_sems.at[j], recv_sem=wo_recv_sems.at[j],
                device_id=xpeer, device_id_type=pl.DeviceIdType.MESH,
            ) for j in range(NW)
        ]
        o_direct = [
            pltpu.make_async_remote_copy(
                src_ref=send_o.at[j // 2, :, pl.ds((j % 2) * SW, SW)],
                dst_ref=recv_o.at[j // 2, :, pl.ds((j % 2) * SW, SW)],
                send_sem=x_send_sems.at[j], recv_sem=o_recv_sems.at[j],
                device_id=xpeer, device_id_type=pl.DeviceIdType.MESH,
            ) for j in range(NW)
        ]

        @pl.when(is_even)
        def _():
            for r in wo_direct:
                r.start()

        @pl.when(jnp.logical_not(is_even))
        def _():
            for r in o_direct:
                r.start()

        for b in range(B):
            mine = ot_vmem[b, :, :, pl.ds(my_x * SH, SH)]
            out_vmem[b, :, :] = lax.dot_general(
                mine.astype(jnp.bfloat16).reshape(K, SH), w, _TN,
                preferred_element_type=jnp.float32)

        wo_fwd = [
            pltpu.make_async_remote_copy(
                src_ref=recv_wo.at[pl.ds(j * KW, KW), :],
                dst_ref=recv_wo.at[pl.ds(j * KW, KW), :],
                send_sem=fwd_send_sems.at[j], recv_sem=wo_recv_sems.at[j],
                device_id=ypart, device_id_type=pl.DeviceIdType.MESH,
            ) for j in range(NW)
        ]
        o_fwd = [
            pltpu.make_async_remote_copy(
                src_ref=recv_o.at[j // 2, :, pl.ds((j % 2) * SW, SW)],
                dst_ref=recv_o.at[j // 2, :, pl.ds((j % 2) * SW, SW)],
                send_sem=fwd_send_sems.at[j], recv_sem=o_recv_sems.at[j],
                device_id=ypart, device_id_type=pl.DeviceIdType.MESH,
            ) for j in range(NW)
        ]

        def tile(b, i):
            out_vmem[b, :, :] += lax.dot_general(
                recv_o[b, pl.ds(i * KC, KC), :],
                recv_wo[pl.ds(i * KC, KC), :],
                _TN, preferred_element_type=jnp.float32)

        def flush(b):
            out_bf[b, :, :] = out_vmem[b, :, :].astype(jnp.bfloat16)
            cp = pltpu.make_async_copy(
                out_bf.at[b], out_hbm.at[b], out_sems.at[b])
            cp.start()
            return cp

        @pl.when(is_even)
        def _():
            for j in range(NW):
                wo_direct[j].wait_recv()
                wo_fwd[j].start()
            for b in range(B):
                o_fwd[2 * b].wait_recv()
                o_fwd[2 * b + 1].wait_recv()
                for i in range(NC):
                    tile(b, i)
                flush(b)

        @pl.when(jnp.logical_not(is_even))
        def _():
            for j in range(NW):
                o_direct[j].wait_recv()
                o_fwd[j].start()
            for i in range(NC):
                wo_fwd[2 * i].wait_recv()
                wo_fwd[2 * i + 1].wait_recv()
                for b in range(B):
                    tile(b, i)
                    if i == NC - 1:
                        flush(b)

        def out_wait(b):
            pltpu.make_async_copy(
                out_bf.at[b], out_hbm.at[b], out_sems.at[b]).wait()

        @pl.when(is_even)
        def _():
            for j in range(NW):
                wo_direct[j].wait_send()
                wo_fwd[j].wait_send()
            for b in range(B):
                out_wait(b)

        @pl.when(jnp.logical_not(is_even))
        def _():
            for j in range(NW):
                o_direct[j].wait_send()
                o_fwd[j].wait_send()
            for b in range(B):
                out_wait(b)

    return pl.pallas_call(
        body,
        out_shape=jax.ShapeDtypeStruct((B, SH, N), jnp.bfloat16),
        in_specs=[
            pl.BlockSpec(memory_space=pl.ANY),
            pl.BlockSpec(memory_space=pl.ANY),
        ],
        out_specs=pl.BlockSpec(memory_space=pl.ANY),
        scratch_shapes=[
            pltpu.VMEM((K, N), jnp.float32),
            pltpu.VMEM((B, H, D, S), jnp.float32),
            pltpu.VMEM((B, SH, N), jnp.float32),
            pltpu.VMEM((B, SH, N), jnp.bfloat16),
            pltpu.VMEM((K, N), jnp.bfloat16),
            pltpu.VMEM((K, N), jnp.bfloat16),
            pltpu.VMEM((B, K, SH), jnp.bfloat16),
            pltpu.VMEM((B, K, SH), jnp.bfloat16),
            pltpu.SemaphoreType.DMA,
            pltpu.SemaphoreType.DMA((B,)),
            pltpu.SemaphoreType.DMA((B,)),
            pltpu.SemaphoreType.DMA((NW,)),
            pltpu.SemaphoreType.DMA((NW,)),
            pltpu.SemaphoreType.DMA((NW,)),
            pltpu.SemaphoreType.DMA((NW,)),
        ],
        compiler_params=pltpu.CompilerParams(collective_id=0),
    )(OT, Wo)
